# baseline (speedup 1.0000x reference)
import jax
import jax.numpy as jnp
from jax.experimental import pallas as pl

_DEPTH = 1000
_ROWS = 16384
_RBLK = 1024
_SUB = _RBLK // 128


def _onehot_body(x_ref, o_ref):
    iota = jax.lax.broadcasted_iota(jnp.int32, (_DEPTH, 128), 0)
    for s in range(_SUB):
        idx = x_ref[s, :].reshape(1, 128)
        o_ref[:, pl.ds(128 * s, 128)] = (iota == idx).astype(jnp.float32)


def kernel(x):
    xi = x.astype(jnp.int32).reshape(_ROWS // 128, 128)
    out = pl.pallas_call(
        _onehot_body,
        grid=(_ROWS // _RBLK,),
        in_specs=[pl.BlockSpec((_SUB, 128), lambda i: (i, 0))],
        out_specs=pl.BlockSpec((_DEPTH, _RBLK), lambda i: (0, i)),
        out_shape=jax.ShapeDtypeStruct((_DEPTH, _ROWS), jnp.float32),
    )(xi)
    return out.T.reshape(_ROWS, 1, _DEPTH)


# speedup vs baseline: 1.0848x; 1.0848x over previous
import jax
import jax.numpy as jnp
from jax.experimental import pallas as pl

_DEPTH = 1000
_ROWS = 16384
_RBLK = 1024


def _onehot_body(x_ref, o_ref):
    idx = x_ref[...]
    iota = jax.lax.broadcasted_iota(jnp.int32, (_DEPTH, _RBLK), 0)
    o_ref[...] = (iota == idx).astype(jnp.float32)


def kernel(x):
    xi = x.astype(jnp.int32).reshape(1, _ROWS)
    out = pl.pallas_call(
        _onehot_body,
        grid=(_ROWS // _RBLK,),
        in_specs=[pl.BlockSpec((1, _RBLK), lambda i: (0, i))],
        out_specs=pl.BlockSpec((_DEPTH, _RBLK), lambda i: (0, i)),
        out_shape=jax.ShapeDtypeStruct((_DEPTH, _ROWS), jnp.float32),
    )(xi)
    return out.T.reshape(_ROWS, 1, _DEPTH)


# speedup vs baseline: 1.0864x; 1.0015x over previous
import jax
import jax.numpy as jnp
from jax.experimental import pallas as pl

_DEPTH = 1000
_ROWS = 16384
_RBLK = 1024


def _onehot_body(x_ref, o_ref):
    j = pl.program_id(0)
    start = pl.multiple_of(j * _RBLK, 128)
    idx = x_ref[:, pl.ds(start, _RBLK)]
    iota = jax.lax.broadcasted_iota(jnp.int32, (_DEPTH, _RBLK), 0)
    o_ref[...] = (iota == idx).astype(jnp.float32)


def wholex_kernel(x):
    xi = x.astype(jnp.int32).reshape(1, _ROWS)
    out = pl.pallas_call(
        _onehot_body,
        grid=(_ROWS // _RBLK,),
        in_specs=[pl.BlockSpec((1, _ROWS), lambda i: (0, 0))],
        out_specs=pl.BlockSpec((_DEPTH, _RBLK), lambda i: (0, i)),
        out_shape=jax.ShapeDtypeStruct((_DEPTH, _ROWS), jnp.float32),
    )(xi)
    return out.T.reshape(_ROWS, 1, _DEPTH)


kernel = wholex_kernel
